# baseline (speedup 1.0000x reference)
import functools

import jax
import jax.numpy as jnp
from jax import lax
from jax.experimental import pallas as pl
from jax.experimental.pallas import tpu as pltpu
from jax.experimental.pallas import tpu_sc as plsc

N = 10000
E = 160000
NC = 2
NS = 16
EPW = E // NS
CH = 125
NCHUNK = EPW // CH
RPT0 = 624
RPT1 = N - RPT0 * (NS - 1)


@functools.lru_cache(maxsize=None)
def _sc_conv(C2):
  mesh = plsc.VectorSubcoreMesh(
      core_axis_name="c", subcore_axis_name="s", num_cores=NC, num_subcores=NS
  )

  def body(h_hbm, src_hbm, dst_hbm, kern_hbm, ksc_hbm, out_hbm,
           srcv, dstv, kv, kscv, rows, rows_b, acc, sem, sem_b, ssem, ssem_b):
    cid = lax.axis_index("c")
    sid = lax.axis_index("s")

    pltpu.sync_copy(src_hbm.at[sid], srcv)
    pltpu.sync_copy(dst_hbm.at[sid], dstv)
    pltpu.sync_copy(kern_hbm.at[sid], kv)
    pltpu.sync_copy(ksc_hbm, kscv)

    def zrow(i, carry):
      for j in range(C2 // 16):
        rows[i, pl.ds(16 * j, 16)] = jnp.zeros((16,), jnp.float32)
      return carry

    lax.fori_loop(0, CH, zrow, 0)
    rbase = sid * RPT0

    def zero_span(base, nrows):
      for t in range(nrows // CH):
        pltpu.sync_copy(rows, acc.at[pl.ds(base + t * CH, CH)])
      rem = nrows % CH
      if rem:
        pltpu.sync_copy(rows.at[pl.ds(0, rem)],
                        acc.at[pl.ds(base + (nrows // CH) * CH, rem)])

    @pl.when(sid == NS - 1)
    def _():
      zero_span(rbase, RPT1)

    @pl.when(sid != NS - 1)
    def _():
      zero_span(rbase, RPT0)

    plsc.subcore_barrier()

    def scale_chunk(buf, g):
      @plsc.parallel_loop(0, CH, unroll=25)
      def _(i):
        k = kv[pl.ds(g * CH + i, 16)][0]
        w = kscv[pl.ds(k, 16)][0]
        for j in range(C2 // 16):
          buf[i, pl.ds(16 * j, 16)] = buf[i, pl.ds(16 * j, 16)] * w

    pltpu.async_copy(h_hbm.at[cid].at[srcv.at[0]], rows, sem)

    def chunk2(k, carry):
      g0 = 2 * k
      g1 = g0 + 1

      @pl.when(k > 0)
      def _():
        pltpu.make_async_copy(rows_b, acc.at[dstv.at[0]], ssem_b).wait()

      pltpu.async_copy(h_hbm.at[cid].at[srcv.at[g1]], rows_b, sem_b)
      pltpu.make_async_copy(h_hbm.at[cid].at[srcv.at[g0]], rows, sem).wait()
      scale_chunk(rows, g0)
      pltpu.async_copy(rows, acc.at[dstv.at[g0]], ssem, add=True)
      pltpu.make_async_copy(h_hbm.at[cid].at[srcv.at[g1]], rows_b, sem_b).wait()
      scale_chunk(rows_b, g1)
      pltpu.make_async_copy(rows, acc.at[dstv.at[0]], ssem).wait()

      @pl.when(k < NCHUNK // 2 - 1)
      def _():
        pltpu.async_copy(h_hbm.at[cid].at[srcv.at[g0 + 2]], rows, sem)

      pltpu.async_copy(rows_b, acc.at[dstv.at[g1]], ssem_b, add=True)
      return carry

    lax.fori_loop(0, NCHUNK // 2, chunk2, 0)
    pltpu.make_async_copy(rows_b, acc.at[dstv.at[0]], ssem_b).wait()
    plsc.subcore_barrier()

    @pl.when(sid == NS - 1)
    def _():
      pltpu.sync_copy(acc.at[pl.ds(rbase, RPT1)],
                      out_hbm.at[cid, pl.ds(rbase, RPT1)])

    @pl.when(sid != NS - 1)
    def _():
      pltpu.sync_copy(acc.at[pl.ds(rbase, RPT0)],
                      out_hbm.at[cid, pl.ds(rbase, RPT0)])

  return pl.kernel(
      body,
      out_type=jax.ShapeDtypeStruct((NC, N, C2), jnp.float32),
      mesh=mesh,
      compiler_params=pltpu.CompilerParams(use_tc_tiling_on_sc=False),
      scratch_types=[
          pltpu.VMEM((NCHUNK, CH), jnp.int32),
          pltpu.VMEM((NCHUNK, CH), jnp.int32),
          pltpu.VMEM((EPW + 16,), jnp.int32),
          pltpu.VMEM((48,), jnp.float32),
          pltpu.VMEM((CH, C2), jnp.float32),
          pltpu.VMEM((CH, C2), jnp.float32),
          pltpu.VMEM_SHARED((N, C2), jnp.float32),
          pltpu.SemaphoreType.DMA,
          pltpu.SemaphoreType.DMA,
          pltpu.SemaphoreType.DMA,
          pltpu.SemaphoreType.DMA,
      ],
  )


BM = 400


def _mm_body(outer, a_ref, h_ref, w_ref, o_ref):
  s = jnp.concatenate([a_ref[0] + h_ref[0], a_ref[1] + h_ref[1]], axis=1)
  if outer:
    r = s[:, 0:1] * w_ref[0:1, :]
  else:
    r = jnp.dot(s.astype(jnp.bfloat16), w_ref[...].astype(jnp.bfloat16),
                preferred_element_type=jnp.float32)
  r = jnp.maximum(r, 0.0)
  co2 = o_ref.shape[2]
  o_ref[0] = r[:, :co2]
  o_ref[1] = r[:, co2:]


def _conv_mm(agg2, h2, w, outer=False):
  c2 = h2.shape[2]
  co = w.shape[1]
  co2 = co // 2
  return pl.pallas_call(
      functools.partial(_mm_body, outer),
      grid=(N // BM,),
      in_specs=[
          pl.BlockSpec((NC, BM, c2), lambda i: (0, i, 0)),
          pl.BlockSpec((NC, BM, c2), lambda i: (0, i, 0)),
          pl.BlockSpec((2 * c2, co), lambda i: (0, 0)),
      ],
      out_specs=pl.BlockSpec((NC, BM, co2), lambda i: (0, i, 0)),
      out_shape=jax.ShapeDtypeStruct((NC, N, co2), jnp.float32),
  )(agg2, h2, w)


def _stats_body(f_ref, o_ref, acc):
  p = pl.program_id(0)
  i = pl.program_id(1)

  @pl.when((p == 0) & (i == 0))
  def _():
    acc[...] = jnp.zeros_like(acc)

  x = f_ref[...]

  @pl.when(p == 0)
  def _():
    acc[0:1, :] += jnp.sum(x, axis=0, keepdims=True)

  @pl.when((p == 0) & (i == N // BM - 1))
  def _():
    acc[0:1, :] = acc[0:1, :] / N

  @pl.when(p == 1)
  def _():
    d = x - acc[0:1, :]
    acc[1:2, :] += jnp.sum(d * d, axis=0, keepdims=True)

  @pl.when((p == 1) & (i == N // BM - 1))
  def _():
    rstd = jax.lax.rsqrt(acc[1:2, :] / N + 1e-4)
    o_ref[...] = jnp.concatenate([acc[0:1, :], rstd], axis=0)


def _bn_stats(feats):
  c = feats.shape[1]
  return pl.pallas_call(
      _stats_body,
      grid=(2, N // BM),
      in_specs=[pl.BlockSpec((BM, c), lambda p, i: (i, 0))],
      out_specs=pl.BlockSpec((2, c), lambda p, i: (0, 0)),
      out_shape=jax.ShapeDtypeStruct((2, c), jnp.float32),
      scratch_shapes=[pltpu.VMEM((2, c), jnp.float32)],
  )(feats)


def _head_body(f_ref, s_ref, g_ref, b_ref, w_ref, bl_ref, o_ref):
  x = f_ref[...]
  xn = (x - s_ref[0:1, :]) * s_ref[1:2, :] * g_ref[...] + b_ref[...]
  xn = jnp.maximum(xn, 0.0)
  o_ref[...] = (
      jnp.dot(xn.astype(jnp.bfloat16), w_ref[...].astype(jnp.bfloat16),
              preferred_element_type=jnp.float32) + bl_ref[...])


def _head(feats, stats, gamma, beta, w_lin, b_lin):
  c = feats.shape[1]
  co = w_lin.shape[1]
  return pl.pallas_call(
      _head_body,
      grid=(N // BM,),
      in_specs=[
          pl.BlockSpec((BM, c), lambda i: (i, 0)),
          pl.BlockSpec((2, c), lambda i: (0, 0)),
          pl.BlockSpec((1, c), lambda i: (0, 0)),
          pl.BlockSpec((1, c), lambda i: (0, 0)),
          pl.BlockSpec((c, co), lambda i: (0, 0)),
          pl.BlockSpec((1, co), lambda i: (0, 0)),
      ],
      out_specs=pl.BlockSpec((BM, co), lambda i: (i, 0)),
      out_shape=jax.ShapeDtypeStruct((N, co), jnp.float32),
  )(feats, stats, gamma, beta, w_lin, b_lin)


def kernel(x, edge_index, edge_kernel, kscale, W_in, W_d1, W_d2, W_d3, W_d4,
           W0a, W0b, W1a, W1b, W2a, W2b, W3a, W3b, W4a, W4b,
           gamma, beta, W_lin, b_lin):
  src = edge_index[0].reshape(NS, NCHUNK, CH)
  dst = edge_index[1].reshape(NS, NCHUNK, CH)
  kern = jnp.pad(edge_kernel.reshape(NS, EPW), ((0, 0), (0, 16)))
  ksc48 = jnp.pad(kscale, (0, 48 - kscale.shape[0]))

  def conv(h2, w, outer=False):
    agg2 = _sc_conv(h2.shape[2])(h2, src, dst, kern, ksc48)
    return _conv_mm(agg2, h2, w, outer=outer)

  xp = jnp.pad(x, ((0, 0), (0, 31))).reshape(N, 2, 16).transpose(1, 0, 2)
  w_in_p = jnp.pad(W_in, ((0, 31), (0, 0)))
  cur = conv(xp, w_in_p, outer=True)

  downs = [None, W_d1, W_d2, W_d3, W_d4]
  rep_ws = [(W0a, W0b), (W1a, W1b), (W2a, W2b), (W3a, W3b), (W4a, W4b)]
  feats = []
  for i in range(5):
    if downs[i] is not None:
      cur = conv(cur, downs[i])
    for w in rep_ws[i]:
      cur = conv(cur, w)
    feats.append(cur)

  cat = jnp.concatenate([jnp.concatenate([f[0], f[1]], axis=1) for f in feats],
                        axis=1)
  stats = _bn_stats(cat)
  return _head(cat, stats, gamma.reshape(1, -1), beta.reshape(1, -1),
               W_lin, b_lin.reshape(1, -1))


# speedup vs baseline: 1.0941x; 1.0941x over previous
import functools

import jax
import jax.numpy as jnp
from jax import lax
from jax.experimental import pallas as pl
from jax.experimental.pallas import tpu as pltpu
from jax.experimental.pallas import tpu_sc as plsc

N = 10000
E = 160000
NC = 2
NS = 16
EPW = E // NS
CH = 125
NCHUNK = EPW // CH
CPB = 2
RPT0 = 624
RPT1 = N - RPT0 * (NS - 1)


@functools.lru_cache(maxsize=None)
def _sc_conv(C2):
  mesh = plsc.VectorSubcoreMesh(
      core_axis_name="c", subcore_axis_name="s", num_cores=NC, num_subcores=NS
  )

  def body(h_hbm, src_hbm, dst_hbm, kern_hbm, ksc_hbm, out_hbm,
           srcv, dstv, kv, kscv, rows, rows_b, acc, sem, sem_b, ssem, ssem_b):
    cid = lax.axis_index("c")
    sid = lax.axis_index("s")

    pltpu.sync_copy(src_hbm.at[sid], srcv)
    pltpu.sync_copy(dst_hbm.at[sid], dstv)
    pltpu.sync_copy(kern_hbm.at[sid], kv)
    pltpu.sync_copy(ksc_hbm, kscv)

    def zrow(i, carry):
      for j in range(C2 // 16):
        rows[i, pl.ds(16 * j, 16)] = jnp.zeros((16,), jnp.float32)
      return carry

    lax.fori_loop(0, CH, zrow, 0)
    rbase = sid * RPT0

    def zero_span(base, nrows):
      for t in range(nrows // CH):
        pltpu.sync_copy(rows.at[pl.ds(0, CH)], acc.at[pl.ds(base + t * CH, CH)])
      rem = nrows % CH
      if rem:
        pltpu.sync_copy(rows.at[pl.ds(0, rem)],
                        acc.at[pl.ds(base + (nrows // CH) * CH, rem)])

    @pl.when(sid == NS - 1)
    def _():
      zero_span(rbase, RPT1)

    @pl.when(sid != NS - 1)
    def _():
      zero_span(rbase, RPT0)

    plsc.subcore_barrier()

    def scale_buf(buf, g):
      @plsc.parallel_loop(0, CPB * CH, unroll=25)
      def _(i):
        k = kv[pl.ds(g * CH + i, 16)][0]
        w = kscv[pl.ds(k, 16)][0]
        for j in range(C2 // 16):
          buf[i, pl.ds(16 * j, 16)] = buf[i, pl.ds(16 * j, 16)] * w

    def gather_buf(buf, g, s):
      for t in range(CPB):
        pltpu.async_copy(h_hbm.at[cid].at[srcv.at[g + t]],
                         buf.at[pl.ds(t * CH, CH)], s)

    def wait_gather_buf(buf, g, s):
      for t in range(CPB):
        pltpu.make_async_copy(h_hbm.at[cid].at[srcv.at[g + t]],
                              buf.at[pl.ds(t * CH, CH)], s).wait()

    def scatter_buf(buf, g, s):
      for t in range(CPB):
        pltpu.async_copy(buf.at[pl.ds(t * CH, CH)],
                         acc.at[dstv.at[g + t]], s, add=True)

    def wait_scatter_buf(buf, s):
      for t in range(CPB):
        pltpu.make_async_copy(buf.at[pl.ds(t * CH, CH)],
                              acc.at[dstv.at[0]], s).wait()

    gather_buf(rows, 0, sem)
    NIT = NCHUNK // (2 * CPB)

    def chunk2(k, carry):
      g0 = 2 * CPB * k
      g1 = g0 + CPB

      @pl.when(k > 0)
      def _():
        wait_scatter_buf(rows_b, ssem_b)

      gather_buf(rows_b, g1, sem_b)
      wait_gather_buf(rows, g0, sem)
      scale_buf(rows, g0)
      scatter_buf(rows, g0, ssem)
      wait_gather_buf(rows_b, g1, sem_b)
      scale_buf(rows_b, g1)
      wait_scatter_buf(rows, ssem)

      @pl.when(k < NIT - 1)
      def _():
        gather_buf(rows, g0 + 2 * CPB, sem)

      scatter_buf(rows_b, g1, ssem_b)
      return carry

    lax.fori_loop(0, NIT, chunk2, 0)
    wait_scatter_buf(rows_b, ssem_b)
    plsc.subcore_barrier()

    @pl.when(sid == NS - 1)
    def _():
      pltpu.sync_copy(acc.at[pl.ds(rbase, RPT1)],
                      out_hbm.at[cid, pl.ds(rbase, RPT1)])

    @pl.when(sid != NS - 1)
    def _():
      pltpu.sync_copy(acc.at[pl.ds(rbase, RPT0)],
                      out_hbm.at[cid, pl.ds(rbase, RPT0)])

  return pl.kernel(
      body,
      out_type=jax.ShapeDtypeStruct((NC, N, C2), jnp.float32),
      mesh=mesh,
      compiler_params=pltpu.CompilerParams(use_tc_tiling_on_sc=False),
      scratch_types=[
          pltpu.VMEM((NCHUNK, CH), jnp.int32),
          pltpu.VMEM((NCHUNK, CH), jnp.int32),
          pltpu.VMEM((EPW + 16,), jnp.int32),
          pltpu.VMEM((48,), jnp.float32),
          pltpu.VMEM((CPB * CH, C2), jnp.float32),
          pltpu.VMEM((CPB * CH, C2), jnp.float32),
          pltpu.VMEM_SHARED((N, C2), jnp.float32),
          pltpu.SemaphoreType.DMA,
          pltpu.SemaphoreType.DMA,
          pltpu.SemaphoreType.DMA,
          pltpu.SemaphoreType.DMA,
      ],
  )


BM = 400


def _mm_body(outer, a_ref, h_ref, w_ref, o_ref):
  s = jnp.concatenate([a_ref[0] + h_ref[0], a_ref[1] + h_ref[1]], axis=1)
  if outer:
    r = s[:, 0:1] * w_ref[0:1, :]
  else:
    r = jnp.dot(s.astype(jnp.bfloat16), w_ref[...].astype(jnp.bfloat16),
                preferred_element_type=jnp.float32)
  r = jnp.maximum(r, 0.0)
  co2 = o_ref.shape[2]
  o_ref[0] = r[:, :co2]
  o_ref[1] = r[:, co2:]


def _conv_mm(agg2, h2, w, outer=False):
  c2 = h2.shape[2]
  co = w.shape[1]
  co2 = co // 2
  return pl.pallas_call(
      functools.partial(_mm_body, outer),
      grid=(N // BM,),
      in_specs=[
          pl.BlockSpec((NC, BM, c2), lambda i: (0, i, 0)),
          pl.BlockSpec((NC, BM, c2), lambda i: (0, i, 0)),
          pl.BlockSpec((2 * c2, co), lambda i: (0, 0)),
      ],
      out_specs=pl.BlockSpec((NC, BM, co2), lambda i: (0, i, 0)),
      out_shape=jax.ShapeDtypeStruct((NC, N, co2), jnp.float32),
  )(agg2, h2, w)


def _stats_body(f_ref, o_ref, acc):
  p = pl.program_id(0)
  i = pl.program_id(1)

  @pl.when((p == 0) & (i == 0))
  def _():
    acc[...] = jnp.zeros_like(acc)

  x = f_ref[...]

  @pl.when(p == 0)
  def _():
    acc[0:1, :] += jnp.sum(x, axis=0, keepdims=True)

  @pl.when((p == 0) & (i == N // BM - 1))
  def _():
    acc[0:1, :] = acc[0:1, :] / N

  @pl.when(p == 1)
  def _():
    d = x - acc[0:1, :]
    acc[1:2, :] += jnp.sum(d * d, axis=0, keepdims=True)

  @pl.when((p == 1) & (i == N // BM - 1))
  def _():
    rstd = jax.lax.rsqrt(acc[1:2, :] / N + 1e-4)
    o_ref[...] = jnp.concatenate([acc[0:1, :], rstd], axis=0)


def _bn_stats(feats):
  c = feats.shape[1]
  return pl.pallas_call(
      _stats_body,
      grid=(2, N // BM),
      in_specs=[pl.BlockSpec((BM, c), lambda p, i: (i, 0))],
      out_specs=pl.BlockSpec((2, c), lambda p, i: (0, 0)),
      out_shape=jax.ShapeDtypeStruct((2, c), jnp.float32),
      scratch_shapes=[pltpu.VMEM((2, c), jnp.float32)],
  )(feats)


def _head_body(f_ref, s_ref, g_ref, b_ref, w_ref, bl_ref, o_ref):
  x = f_ref[...]
  xn = (x - s_ref[0:1, :]) * s_ref[1:2, :] * g_ref[...] + b_ref[...]
  xn = jnp.maximum(xn, 0.0)
  o_ref[...] = (
      jnp.dot(xn.astype(jnp.bfloat16), w_ref[...].astype(jnp.bfloat16),
              preferred_element_type=jnp.float32) + bl_ref[...])


def _head(feats, stats, gamma, beta, w_lin, b_lin):
  c = feats.shape[1]
  co = w_lin.shape[1]
  return pl.pallas_call(
      _head_body,
      grid=(N // BM,),
      in_specs=[
          pl.BlockSpec((BM, c), lambda i: (i, 0)),
          pl.BlockSpec((2, c), lambda i: (0, 0)),
          pl.BlockSpec((1, c), lambda i: (0, 0)),
          pl.BlockSpec((1, c), lambda i: (0, 0)),
          pl.BlockSpec((c, co), lambda i: (0, 0)),
          pl.BlockSpec((1, co), lambda i: (0, 0)),
      ],
      out_specs=pl.BlockSpec((BM, co), lambda i: (i, 0)),
      out_shape=jax.ShapeDtypeStruct((N, co), jnp.float32),
  )(feats, stats, gamma, beta, w_lin, b_lin)


def kernel(x, edge_index, edge_kernel, kscale, W_in, W_d1, W_d2, W_d3, W_d4,
           W0a, W0b, W1a, W1b, W2a, W2b, W3a, W3b, W4a, W4b,
           gamma, beta, W_lin, b_lin):
  src = edge_index[0].reshape(NS, NCHUNK, CH)
  dst = edge_index[1].reshape(NS, NCHUNK, CH)
  kern = jnp.pad(edge_kernel.reshape(NS, EPW), ((0, 0), (0, 16)))
  ksc48 = jnp.pad(kscale, (0, 48 - kscale.shape[0]))

  def conv(h2, w, outer=False):
    agg2 = _sc_conv(h2.shape[2])(h2, src, dst, kern, ksc48)
    return _conv_mm(agg2, h2, w, outer=outer)

  xp = jnp.pad(x, ((0, 0), (0, 31))).reshape(N, 2, 16).transpose(1, 0, 2)
  w_in_p = jnp.pad(W_in, ((0, 31), (0, 0)))
  cur = conv(xp, w_in_p, outer=True)

  downs = [None, W_d1, W_d2, W_d3, W_d4]
  rep_ws = [(W0a, W0b), (W1a, W1b), (W2a, W2b), (W3a, W3b), (W4a, W4b)]
  feats = []
  for i in range(5):
    if downs[i] is not None:
      cur = conv(cur, downs[i])
    for w in rep_ws[i]:
      cur = conv(cur, w)
    feats.append(cur)

  cat = jnp.concatenate([jnp.concatenate([f[0], f[1]], axis=1) for f in feats],
                        axis=1)
  stats = _bn_stats(cat)
  return _head(cat, stats, gamma.reshape(1, -1), beta.reshape(1, -1),
               W_lin, b_lin.reshape(1, -1))
